# baseline (speedup 1.0000x reference)
import jax
import jax.numpy as jnp
from jax import lax
from jax.experimental import pallas as pl
from jax.experimental.pallas import tpu as pltpu
from jax.experimental.pallas import tpu_sc as plsc

N_CAT = 26
N_NUM = 13
B = 16384
VOCAB = 100001
D = 32

NC = 2
NS = 16
NW = NC * NS
Q = 4096
NQ = B // Q
HB = 8192
VH = 50048
VL = VOCAB - VH


def _body(tabT, cat, num, wx, bx, out,
          pla, plb, idxh, ob, nb0, nb1, wall, ball,
          psa, psb, isem, wsem, nsem, nwsem):
    wid = lax.axis_index("s") * NC + lax.axis_index("c")
    s = lax.axis_index("s")
    d = wid

    nbb = [nb0, nb1]
    nwdesc = [None, None]

    def fields(j):
        return lax.rem(s + j, N_CAT)

    pda = pltpu.async_copy(tabT.at[fields(0), pl.ds(d, 1), pl.ds(0, VH)],
                           pla, psa)
    pdb = pltpu.async_copy(tabT.at[fields(0), pl.ds(d, 1), pl.ds(VH, VL)],
                           plb, psb)

    for n in range(N_NUM):
        pltpu.sync_copy(wx.at[pl.ds(n * D + d, 1)], wall.at[pl.ds(n, 1)])
        pltpu.sync_copy(bx.at[pl.ds(n * D + d, 1)], ball.at[pl.ds(n, 1)])

    def process_num(n):
        wv = wall[n, :]
        bv = ball[n, :]
        row = (N_CAT + n) * D + d
        for q in range(NQ):
            bidx = q % 2
            nb = nbb[bidx]
            if nwdesc[bidx] is not None:
                nwdesc[bidx].wait()
            pltpu.async_copy(num.at[pl.ds(n, 1), pl.ds(q * Q, Q)],
                             nb, nsem).wait()

            @plsc.parallel_loop(0, Q // 16, unroll=4)
            def _(i, nb=nb, wv=wv, bv=bv):
                sl = pl.ds(i * 16, 16)
                nb[0, sl] = nb[0, sl] * wv + bv

            nwdesc[bidx] = pltpu.async_copy(
                nb, out.at[pl.ds(row, 1), pl.ds(q * Q, Q)], nwsem)

    wdesc = [None, None]
    for j in range(N_CAT):
        fe = fields(j)
        row = fe * D + d
        if j % 2 == 0 and j // 2 < N_NUM:
            process_num(j // 2)
        pda.wait()
        for h in range(2):
            pltpu.sync_copy(cat.at[pl.ds(fe, 1), pl.ds(h * HB, HB)], idxh)
            if wdesc[h] is not None:
                wdesc[h].wait()

            @plsc.parallel_loop(0, HB // 16, unroll=4)
            def _(i, ob=ob):
                sl = pl.ds(i * 16, 16)
                iv = idxh[0, sl]
                ob[0, sl] = plsc.load_gather(pla.at[0], [iv], mask=iv < VH)

            if h == 1:
                if j + 1 < N_CAT:
                    pda = pltpu.async_copy(
                        tabT.at[fields(j + 1), pl.ds(d, 1), pl.ds(0, VH)],
                        pla, psa)
            if h == 0:
                pdb.wait()

            @plsc.parallel_loop(0, HB // 16, unroll=4)
            def _(i, ob=ob):
                sl = pl.ds(i * 16, 16)
                iv = idxh[0, sl]
                m = iv >= VH
                vb = plsc.load_gather(plb.at[0], [iv - VH], mask=m)
                ob[0, sl] = jnp.where(m, vb, ob[0, sl])

            wdesc[h] = pltpu.async_copy(
                ob, out.at[pl.ds(row, 1), pl.ds(h * HB, HB)], wsem)
        if j + 1 < N_CAT:
            pdb = pltpu.async_copy(
                tabT.at[fields(j + 1), pl.ds(d, 1), pl.ds(VH, VL)],
                plb, psb)

    for dd in wdesc:
        if dd is not None:
            dd.wait()
    for dd in nwdesc:
        if dd is not None:
            dd.wait()


@jax.jit
def _run(tabT, cat, num, wx, bx):
    mesh = plsc.VectorSubcoreMesh(core_axis_name="c", subcore_axis_name="s",
                                  num_cores=NC, num_subcores=NS)
    return pl.kernel(
        _body,
        out_type=jax.ShapeDtypeStruct(((N_CAT + N_NUM) * D, B), jnp.float32),
        mesh=mesh,
        compiler_params=pltpu.CompilerParams(needs_layout_passes=False),
        scratch_types=[
            pltpu.VMEM((1, VH), jnp.float32),
            pltpu.VMEM((1, VL), jnp.float32),
            pltpu.VMEM((1, HB), jnp.int32),
            pltpu.VMEM((1, HB), jnp.float32),
            pltpu.VMEM((1, Q), jnp.float32),
            pltpu.VMEM((1, Q), jnp.float32),
            pltpu.VMEM((N_NUM, 16), jnp.float32),
            pltpu.VMEM((N_NUM, 16), jnp.float32),
            pltpu.SemaphoreType.DMA,
            pltpu.SemaphoreType.DMA,
            pltpu.SemaphoreType.DMA,
            pltpu.SemaphoreType.DMA,
            pltpu.SemaphoreType.DMA,
            pltpu.SemaphoreType.DMA,
        ],
    )(tabT, cat, num, wx, bx)


def kernel(cat_features, num_features, tables, W, b):
    tabT = jnp.transpose(tables, (0, 2, 1))
    wx = jnp.broadcast_to(W.reshape(N_NUM * D)[:, None], (N_NUM * D, 16))
    bx = jnp.broadcast_to(b.reshape(N_NUM * D)[:, None], (N_NUM * D, 16))
    out_dm = _run(tabT, cat_features, num_features, wx, bx)
    return out_dm.T


# speedup vs baseline: 1.2429x; 1.2429x over previous
import jax
import jax.numpy as jnp
from jax import lax
from jax.experimental import pallas as pl
from jax.experimental.pallas import tpu as pltpu
from jax.experimental.pallas import tpu_sc as plsc

N_CAT = 26
N_NUM = 13
B = 16384
VOCAB = 100001
D = 32

NC = 2
NS = 16
NW = NC * NS
Q = 4096
NQ = B // Q


def _body(tabT, cat, num, wx, bx, out,
          plane, idx0, idx1, ob0, ob1, nb0, nb1, wall, ball,
          psem, psem2, isem, wsem, nsem, nwsem):
    wid = lax.axis_index("s") * NC + lax.axis_index("c")
    s = lax.axis_index("s")
    d = wid

    idxb = [idx0, idx1]
    obb = [ob0, ob1]
    nbb = [nb0, nb1]
    wdesc = [None, None]
    nwdesc = [None, None]

    pd = pltpu.async_copy(tabT.at[lax.div(wid * N_CAT, NW), pl.ds(d, 1)],
                          plane, psem)

    for n in range(N_NUM):
        pltpu.sync_copy(wx.at[pl.ds(n * D + d, 1)], wall.at[pl.ds(n, 1)])
        pltpu.sync_copy(bx.at[pl.ds(n * D + d, 1)], ball.at[pl.ds(n, 1)])

    def process_num(n):
        wv = wall[n, :]
        bv = ball[n, :]
        row = (N_CAT + n) * D + d
        for q in range(NQ):
            bidx = q % 2
            nb = nbb[bidx]
            if nwdesc[bidx] is not None:
                nwdesc[bidx].wait()
            pltpu.async_copy(num.at[pl.ds(n, 1), pl.ds(q * Q, Q)],
                             nb, nsem).wait()

            @plsc.parallel_loop(0, Q // 16, unroll=8)
            def _(i, nb=nb, wv=wv, bv=bv):
                sl = pl.ds(i * 16, 16)
                nb[0, sl] = nb[0, sl] * wv + bv

            nwdesc[bidx] = pltpu.async_copy(
                nb, out.at[pl.ds(row, 1), pl.ds(q * Q, Q)], nwsem)

    phase = lax.div(wid * N_CAT, NW)
    for j in range(N_CAT):
        f = j
        fe = lax.rem(phase + j, N_CAT)
        if f > 0:
            pd = pltpu.async_copy(tabT.at[fe, pl.ds(d, 1)], plane, psem)
        idesc = [pltpu.async_copy(cat.at[pl.ds(fe, 1), pl.ds(q * Q, Q)],
                                  idxb[q], isem)
                 for q in range(2)]
        if f % 2 == 0 and f // 2 < N_NUM:
            process_num(f // 2)
        pd.wait()
        row = fe * D + d
        for q in range(NQ):
            bidx = q % 2
            idesc[bidx].wait()
            ob = obb[bidx]
            if wdesc[bidx] is not None:
                wdesc[bidx].wait()

            @plsc.parallel_loop(0, Q // 16, unroll=8)
            def _(i, ob=ob, idxr=idxb[bidx]):
                iv = idxr[0, pl.ds(i * 16, 16)]
                ob[0, pl.ds(i * 16, 16)] = plsc.load_gather(plane.at[0], [iv])

            wdesc[bidx] = pltpu.async_copy(
                ob, out.at[pl.ds(row, 1), pl.ds(q * Q, Q)], wsem)
            if q + 2 < NQ:
                idesc[bidx] = pltpu.async_copy(
                    cat.at[pl.ds(fe, 1), pl.ds((q + 2) * Q, Q)],
                    idxb[bidx], isem)

    for dd in wdesc + nwdesc:
        if dd is not None:
            dd.wait()


@jax.jit
def _run(tabT, cat, num, wx, bx):
    mesh = plsc.VectorSubcoreMesh(core_axis_name="c", subcore_axis_name="s",
                                  num_cores=NC, num_subcores=NS)
    return pl.kernel(
        _body,
        out_type=jax.ShapeDtypeStruct(((N_CAT + N_NUM) * D, B), jnp.float32),
        mesh=mesh,
        compiler_params=pltpu.CompilerParams(needs_layout_passes=False),
        scratch_types=[
            pltpu.VMEM((1, VOCAB), jnp.float32),
            pltpu.VMEM((1, Q), jnp.int32),
            pltpu.VMEM((1, Q), jnp.int32),
            pltpu.VMEM((1, Q), jnp.float32),
            pltpu.VMEM((1, Q), jnp.float32),
            pltpu.VMEM((1, Q), jnp.float32),
            pltpu.VMEM((1, Q), jnp.float32),
            pltpu.VMEM((N_NUM, 16), jnp.float32),
            pltpu.VMEM((N_NUM, 16), jnp.float32),
            pltpu.SemaphoreType.DMA,
            pltpu.SemaphoreType.DMA,
            pltpu.SemaphoreType.DMA,
            pltpu.SemaphoreType.DMA,
            pltpu.SemaphoreType.DMA,
            pltpu.SemaphoreType.DMA,
        ],
    )(tabT, cat, num, wx, bx)


def kernel(cat_features, num_features, tables, W, b):
    tabT = jnp.transpose(tables, (0, 2, 1))
    wx = jnp.broadcast_to(W.reshape(N_NUM * D)[:, None], (N_NUM * D, 16))
    bx = jnp.broadcast_to(b.reshape(N_NUM * D)[:, None], (N_NUM * D, 16))
    out_dm = _run(tabT, cat_features, num_features, wx, bx)
    return out_dm.T


# speedup vs baseline: 1.2970x; 1.0436x over previous
import jax
import jax.numpy as jnp
from jax import lax
from jax.experimental import pallas as pl
from jax.experimental.pallas import tpu as pltpu
from jax.experimental.pallas import tpu_sc as plsc

N_CAT = 26
N_NUM = 13
B = 16384
VOCAB = 100001
D = 32

NC = 2
NS = 16
NW = NC * NS
Q = 4096
NQ = B // Q


def _body(tabT, cat, num, wx, bx, out,
          plane, idx0, idx1, ob0, ob1, nb0, nb1, wall, ball,
          psem, psem2, isem, wsem, nsem, nwsem):
    wid = lax.axis_index("s") * NC + lax.axis_index("c")
    s = lax.axis_index("s")
    d = wid

    idxb = [idx0, idx1]
    obb = [ob0, ob1]
    nbb = [nb0, nb1]
    wdesc = [None, None]
    nwdesc = [None, None]

    pd = pltpu.async_copy(tabT.at[lax.div(wid * N_CAT, NW), pl.ds(d, 1)],
                          plane, psem)

    for n in range(N_NUM):
        pltpu.sync_copy(wx.at[pl.ds(n * D + d, 1)], wall.at[pl.ds(n, 1)])
        pltpu.sync_copy(bx.at[pl.ds(n * D + d, 1)], ball.at[pl.ds(n, 1)])

    def process_num_quarter(t):
        n, q = t // NQ, t % NQ
        wv = wall[n, :]
        bv = ball[n, :]
        row = (N_CAT + n) * D + d
        bidx = t % 2
        nb = nbb[bidx]
        if nwdesc[bidx] is not None:
            nwdesc[bidx].wait()
        pltpu.async_copy(num.at[pl.ds(n, 1), pl.ds(q * Q, Q)],
                         nb, nsem).wait()

        @plsc.parallel_loop(0, Q // 16, unroll=4)
        def _(i, nb=nb, wv=wv, bv=bv):
            sl = pl.ds(i * 16, 16)
            nb[0, sl] = nb[0, sl] * wv + bv

        nwdesc[bidx] = pltpu.async_copy(
            nb, out.at[pl.ds(row, 1), pl.ds(q * Q, Q)], nwsem)

    phase = lax.div(wid * N_CAT, NW)
    for j in range(N_CAT):
        f = j
        fe = lax.rem(phase + j, N_CAT)
        if f > 0:
            pd = pltpu.async_copy(tabT.at[fe, pl.ds(d, 1)], plane, psem)
        idesc = [pltpu.async_copy(cat.at[pl.ds(fe, 1), pl.ds(q * Q, Q)],
                                  idxb[q], isem)
                 for q in range(2)]
        for t in (2 * f, 2 * f + 1):
            if t < N_NUM * NQ:
                process_num_quarter(t)
        pd.wait()
        row = fe * D + d
        for q in range(NQ):
            bidx = q % 2
            idesc[bidx].wait()
            ob = obb[bidx]
            if wdesc[bidx] is not None:
                wdesc[bidx].wait()

            @plsc.parallel_loop(0, Q // 16, unroll=8)
            def _(i, ob=ob, idxr=idxb[bidx]):
                iv = idxr[0, pl.ds(i * 16, 16)]
                ob[0, pl.ds(i * 16, 16)] = plsc.load_gather(plane.at[0], [iv])

            wdesc[bidx] = pltpu.async_copy(
                ob, out.at[pl.ds(row, 1), pl.ds(q * Q, Q)], wsem)
            if q + 2 < NQ:
                idesc[bidx] = pltpu.async_copy(
                    cat.at[pl.ds(fe, 1), pl.ds((q + 2) * Q, Q)],
                    idxb[bidx], isem)

    for dd in wdesc + nwdesc:
        if dd is not None:
            dd.wait()


@jax.jit
def _run(tabT, cat, num, wx, bx):
    mesh = plsc.VectorSubcoreMesh(core_axis_name="c", subcore_axis_name="s",
                                  num_cores=NC, num_subcores=NS)
    return pl.kernel(
        _body,
        out_type=jax.ShapeDtypeStruct(((N_CAT + N_NUM) * D, B), jnp.float32),
        mesh=mesh,
        compiler_params=pltpu.CompilerParams(needs_layout_passes=False),
        scratch_types=[
            pltpu.VMEM((1, VOCAB), jnp.float32),
            pltpu.VMEM((1, Q), jnp.int32),
            pltpu.VMEM((1, Q), jnp.int32),
            pltpu.VMEM((1, Q), jnp.float32),
            pltpu.VMEM((1, Q), jnp.float32),
            pltpu.VMEM((1, Q), jnp.float32),
            pltpu.VMEM((1, Q), jnp.float32),
            pltpu.VMEM((N_NUM, 16), jnp.float32),
            pltpu.VMEM((N_NUM, 16), jnp.float32),
            pltpu.SemaphoreType.DMA,
            pltpu.SemaphoreType.DMA,
            pltpu.SemaphoreType.DMA,
            pltpu.SemaphoreType.DMA,
            pltpu.SemaphoreType.DMA,
            pltpu.SemaphoreType.DMA,
        ],
    )(tabT, cat, num, wx, bx)


def kernel(cat_features, num_features, tables, W, b):
    tabT = jnp.transpose(tables, (0, 2, 1))
    wx = jnp.broadcast_to(W.reshape(N_NUM * D)[:, None], (N_NUM * D, 16))
    bx = jnp.broadcast_to(b.reshape(N_NUM * D)[:, None], (N_NUM * D, 16))
    out_dm = _run(tabT, cat_features, num_features, wx, bx)
    return out_dm.T
